# Initial kernel scaffold; baseline (speedup 1.0000x reference)
import jax
import jax.numpy as jnp
from jax.experimental import pallas as pl


def kernel(x, adj, ws1, wn1, b1, ws2, wn2, b2, wa1, ba1, wa2, ba2, wb1, bb1, wb2, bb2):
    raise NotImplementedError("write your pallas kernel here")


# speedup vs baseline: 1.0633x; 1.0633x over previous
import functools

import jax
import jax.numpy as jnp
from jax.experimental import pallas as pl
from jax.experimental.pallas import tpu as pltpu

_TILE = 64
_GROUP = 8


def _gcn_kernel(x_ref, a_ref, w1_ref, w2_ref, wh_ref, wf_ref,
                b1_ref, b2_ref, bh_ref, bf_ref, oa_ref, ob_ref):
    t, n, f_in = x_ref.shape
    r = t * n
    p = _GROUP * n
    n_groups = t // _GROUP

    xb = x_ref[...].reshape(r, f_in)
    a = a_ref[...].reshape(r, n)

    deg = jnp.sum(a.astype(jnp.float32), axis=1, keepdims=True)
    rnorm = jnp.where(deg > 0.0, 1.0 / deg, 0.0).astype(jnp.bfloat16)
    an = a * rnorm

    rows = jax.lax.broadcasted_iota(jnp.int32, (p, p), 0) // n
    cols = jax.lax.broadcasted_iota(jnp.int32, (p, p), 1) // n
    same = (rows == cols).astype(jnp.bfloat16)

    blocks = []
    for gi in range(n_groups):
        g = an[gi * p:(gi + 1) * p, :]
        blocks.append(jnp.concatenate([g] * _GROUP, axis=1) * same)

    def agg(feat_b):
        outs = [jnp.dot(blocks[gi], feat_b[gi * p:(gi + 1) * p, :],
                        preferred_element_type=jnp.float32)
                for gi in range(n_groups)]
        return jnp.concatenate(outs, axis=0)

    y1 = jnp.dot(xb, w1_ref[...], preferred_element_type=jnp.float32)
    n1 = agg(y1[:, f_in:].astype(jnp.bfloat16))
    h1 = jnp.maximum(y1[:, :f_in] + n1 + b1_ref[...], 0.0).astype(jnp.bfloat16)

    y2 = jnp.dot(h1, w2_ref[...], preferred_element_type=jnp.float32)
    n2 = agg(y2[:, f_in:].astype(jnp.bfloat16))
    h2 = jnp.maximum(y2[:, :f_in] + n2 + b2_ref[...], 0.0).astype(jnp.bfloat16)

    u = jnp.dot(h2, wh_ref[...], preferred_element_type=jnp.float32) + bh_ref[...]
    ub = jnp.maximum(u, 0.0).astype(jnp.bfloat16)
    fin = (jnp.dot(ub, wf_ref[...], preferred_element_type=jnp.float32)
           + bf_ref[...])
    oa_ref[...] = fin[:, 0:2].reshape(oa_ref.shape)
    ob_ref[...] = fin[:, 2:3].reshape(ob_ref.shape)


@functools.partial(jax.jit, static_argnames=())
def _forward(x, adj, w1, w2, wh, wf, b1, b2, bh, bf):
    b, n, f_in = x.shape
    t = _TILE
    return pl.pallas_call(
        _gcn_kernel,
        out_shape=(jax.ShapeDtypeStruct((b, n, 2), jnp.float32),
                   jax.ShapeDtypeStruct((b, n, 1), jnp.float32)),
        grid=(b // t,),
        in_specs=[
            pl.BlockSpec((t, n, f_in), lambda i: (i, 0, 0)),
            pl.BlockSpec((t, n, n), lambda i: (i, 0, 0)),
            pl.BlockSpec(w1.shape, lambda i: (0, 0)),
            pl.BlockSpec(w2.shape, lambda i: (0, 0)),
            pl.BlockSpec(wh.shape, lambda i: (0, 0)),
            pl.BlockSpec(wf.shape, lambda i: (0, 0)),
            pl.BlockSpec(b1.shape, lambda i: (0, 0)),
            pl.BlockSpec(b2.shape, lambda i: (0, 0)),
            pl.BlockSpec(bh.shape, lambda i: (0, 0)),
            pl.BlockSpec(bf.shape, lambda i: (0, 0)),
        ],
        out_specs=[
            pl.BlockSpec((t, n, 2), lambda i: (i, 0, 0)),
            pl.BlockSpec((t, n, 1), lambda i: (i, 0, 0)),
        ],
        compiler_params=pltpu.CompilerParams(
            dimension_semantics=("parallel",),
            vmem_limit_bytes=64 * 1024 * 1024,
        ),
    )(x, adj, w1, w2, wh, wf, b1, b2, bh, bf)


def kernel(x, adj, ws1, wn1, b1, ws2, wn2, b2,
           wa1, ba1, wa2, ba2, wb1, bb1, wb2, bb2):
    bf16 = jnp.bfloat16
    w1 = jnp.concatenate([ws1, wn1], axis=1).astype(bf16)
    w2 = jnp.concatenate([ws2, wn2], axis=1).astype(bf16)
    wh = jnp.concatenate([wa1, wb1], axis=1).astype(bf16)
    h = wa2.shape[0]
    zf = jnp.zeros((h, 1), jnp.float32)
    zt = jnp.zeros((h, 2), jnp.float32)
    wf = jnp.concatenate([
        jnp.concatenate([wa2, zf], axis=1),
        jnp.concatenate([zt, wb2], axis=1),
    ], axis=0).astype(bf16)
    bh = jnp.concatenate([ba1, bb1], axis=1)
    bf_ = jnp.concatenate([ba2, bb2], axis=1)
    oa, ob = _forward(x.astype(bf16), adj.astype(bf16),
                      w1, w2, wh, wf, b1, b2, bh, bf_)
    return {"am1-charges": oa, "am1-wbo-like": ob}


# speedup vs baseline: 1.1052x; 1.0395x over previous
import functools

import jax
import jax.numpy as jnp
from jax.experimental import pallas as pl
from jax.experimental.pallas import tpu as pltpu

_TILE = 128
_GROUP = 8


def _gcn_kernel(x_ref, a_ref, w1_ref, w2_ref, wh_ref, wf_ref,
                b1_ref, b2_ref, bh_ref, bf_ref, oa_ref, ob_ref):
    t, n, f_in = x_ref.shape
    r = t * n
    p = _GROUP * n
    n_groups = t // _GROUP

    xb = x_ref[...].reshape(r, f_in)
    a = a_ref[...].reshape(r, n)

    deg = jnp.sum(a, axis=1, keepdims=True)
    rnorm = jnp.where(deg > 0.0, 1.0 / deg, 0.0)
    an = a * rnorm

    rows = jax.lax.broadcasted_iota(jnp.int32, (p, p), 0) // n
    cols = jax.lax.broadcasted_iota(jnp.int32, (p, p), 1) // n
    same = (rows == cols).astype(jnp.float32)

    blocks = []
    for gi in range(n_groups):
        g = an[gi * p:(gi + 1) * p, :]
        blocks.append(jnp.concatenate([g] * _GROUP, axis=1) * same)

    def agg(feat):
        outs = [jnp.dot(blocks[gi], feat[gi * p:(gi + 1) * p, :],
                        preferred_element_type=jnp.float32)
                for gi in range(n_groups)]
        return jnp.concatenate(outs, axis=0)

    y1 = jnp.dot(xb, w1_ref[...], preferred_element_type=jnp.float32)
    n1 = agg(y1[:, f_in:])
    h1 = jnp.maximum(y1[:, :f_in] + n1 + b1_ref[...], 0.0)

    y2 = jnp.dot(h1, w2_ref[...], preferred_element_type=jnp.float32)
    n2 = agg(y2[:, f_in:])
    h2 = jnp.maximum(y2[:, :f_in] + n2 + b2_ref[...], 0.0)

    u = jnp.dot(h2, wh_ref[...], preferred_element_type=jnp.float32) + bh_ref[...]
    ub = jnp.maximum(u, 0.0)
    fin = (jnp.dot(ub, wf_ref[...], preferred_element_type=jnp.float32)
           + bf_ref[...])
    oa_ref[...] = fin[:, 0:2].reshape(oa_ref.shape)
    ob_ref[...] = fin[:, 2:3].reshape(ob_ref.shape)


@functools.partial(jax.jit, static_argnames=())
def _forward(x, adj, w1, w2, wh, wf, b1, b2, bh, bf):
    b, n, f_in = x.shape
    t = _TILE
    return pl.pallas_call(
        _gcn_kernel,
        out_shape=(jax.ShapeDtypeStruct((b, n, 2), jnp.float32),
                   jax.ShapeDtypeStruct((b, n, 1), jnp.float32)),
        grid=(b // t,),
        in_specs=[
            pl.BlockSpec((t, n, f_in), lambda i: (i, 0, 0)),
            pl.BlockSpec((t, n, n), lambda i: (i, 0, 0)),
            pl.BlockSpec(w1.shape, lambda i: (0, 0)),
            pl.BlockSpec(w2.shape, lambda i: (0, 0)),
            pl.BlockSpec(wh.shape, lambda i: (0, 0)),
            pl.BlockSpec(wf.shape, lambda i: (0, 0)),
            pl.BlockSpec(b1.shape, lambda i: (0, 0)),
            pl.BlockSpec(b2.shape, lambda i: (0, 0)),
            pl.BlockSpec(bh.shape, lambda i: (0, 0)),
            pl.BlockSpec(bf.shape, lambda i: (0, 0)),
        ],
        out_specs=[
            pl.BlockSpec((t, n, 2), lambda i: (i, 0, 0)),
            pl.BlockSpec((t, n, 1), lambda i: (i, 0, 0)),
        ],
        compiler_params=pltpu.CompilerParams(
            dimension_semantics=("parallel",),
            vmem_limit_bytes=64 * 1024 * 1024,
        ),
    )(x, adj, w1, w2, wh, wf, b1, b2, bh, bf)


def kernel(x, adj, ws1, wn1, b1, ws2, wn2, b2,
           wa1, ba1, wa2, ba2, wb1, bb1, wb2, bb2):
    w1 = jnp.concatenate([ws1, wn1], axis=1)
    w2 = jnp.concatenate([ws2, wn2], axis=1)
    wh = jnp.concatenate([wa1, wb1], axis=1)
    h = wa2.shape[0]
    zf = jnp.zeros((h, 1), jnp.float32)
    zt = jnp.zeros((h, 2), jnp.float32)
    wf = jnp.concatenate([
        jnp.concatenate([wa2, zf], axis=1),
        jnp.concatenate([zt, wb2], axis=1),
    ], axis=0)
    bh = jnp.concatenate([ba1, bb1], axis=1)
    bf_ = jnp.concatenate([ba2, bb2], axis=1)
    oa, ob = _forward(x, adj, w1, w2, wh, wf, b1, b2, bh, bf_)
    return {"am1-charges": oa, "am1-wbo-like": ob}


# speedup vs baseline: 1.8374x; 1.6624x over previous
import functools

import jax
import jax.numpy as jnp
from jax.experimental import pallas as pl
from jax.experimental.pallas import tpu as pltpu

_TILE = 128
_GROUP = 8


def _gcn_kernel(x_ref, a_ref, w1_ref, w2_ref, wh_ref, wf_ref,
                b1_ref, b2_ref, bh_ref, bf_ref, oa_ref, ob_ref):
    t, n, f_in = x_ref.shape
    r = t * n
    p = _GROUP * n
    n_groups = t // _GROUP

    xb = x_ref[...].reshape(r, f_in)
    a = a_ref[...].reshape(r, n)

    deg = jnp.sum(a, axis=1, keepdims=True)
    rnorm = jnp.where(deg > 0.0, 1.0 / deg, 0.0)
    an = a * rnorm

    rows = jax.lax.broadcasted_iota(jnp.int32, (p, p), 0) // n
    cols = jax.lax.broadcasted_iota(jnp.int32, (p, p), 1) // n
    same = (rows == cols).astype(jnp.float32)

    tj = jax.lax.broadcasted_iota(jnp.int32, (n, p), 0)
    tc = jax.lax.broadcasted_iota(jnp.int32, (n, p), 1)
    tile_mat = (tc % n == tj).astype(jnp.float32)

    blocks = []
    for gi in range(n_groups):
        g = an[gi * p:(gi + 1) * p, :]
        rep = jnp.dot(g, tile_mat, preferred_element_type=jnp.float32)
        blocks.append(rep * same)

    def agg(feat):
        outs = [jnp.dot(blocks[gi], feat[gi * p:(gi + 1) * p, :],
                        preferred_element_type=jnp.float32)
                for gi in range(n_groups)]
        return jnp.concatenate(outs, axis=0)

    y1 = jnp.dot(xb, w1_ref[...], preferred_element_type=jnp.float32)
    n1 = agg(y1[:, f_in:])
    h1 = jnp.maximum(y1[:, :f_in] + n1 + b1_ref[...], 0.0)

    y2 = jnp.dot(h1, w2_ref[...], preferred_element_type=jnp.float32)
    n2 = agg(y2[:, f_in:])
    h2 = jnp.maximum(y2[:, :f_in] + n2 + b2_ref[...], 0.0)

    u = jnp.dot(h2, wh_ref[...], preferred_element_type=jnp.float32) + bh_ref[...]
    ub = jnp.maximum(u, 0.0)
    fin = (jnp.dot(ub, wf_ref[...], preferred_element_type=jnp.float32)
           + bf_ref[...])
    oa_ref[...] = fin[:, 0:2].reshape(oa_ref.shape)
    ob_ref[...] = fin[:, 2:3].reshape(ob_ref.shape)


@functools.partial(jax.jit, static_argnames=())
def _forward(x, adj, w1, w2, wh, wf, b1, b2, bh, bf):
    b, n, f_in = x.shape
    t = _TILE
    return pl.pallas_call(
        _gcn_kernel,
        out_shape=(jax.ShapeDtypeStruct((b, n, 2), jnp.float32),
                   jax.ShapeDtypeStruct((b, n, 1), jnp.float32)),
        grid=(b // t,),
        in_specs=[
            pl.BlockSpec((t, n, f_in), lambda i: (i, 0, 0)),
            pl.BlockSpec((t, n, n), lambda i: (i, 0, 0)),
            pl.BlockSpec(w1.shape, lambda i: (0, 0)),
            pl.BlockSpec(w2.shape, lambda i: (0, 0)),
            pl.BlockSpec(wh.shape, lambda i: (0, 0)),
            pl.BlockSpec(wf.shape, lambda i: (0, 0)),
            pl.BlockSpec(b1.shape, lambda i: (0, 0)),
            pl.BlockSpec(b2.shape, lambda i: (0, 0)),
            pl.BlockSpec(bh.shape, lambda i: (0, 0)),
            pl.BlockSpec(bf.shape, lambda i: (0, 0)),
        ],
        out_specs=[
            pl.BlockSpec((t, n, 2), lambda i: (i, 0, 0)),
            pl.BlockSpec((t, n, 1), lambda i: (i, 0, 0)),
        ],
        compiler_params=pltpu.CompilerParams(
            dimension_semantics=("parallel",),
            vmem_limit_bytes=64 * 1024 * 1024,
        ),
    )(x, adj, w1, w2, wh, wf, b1, b2, bh, bf)


def kernel(x, adj, ws1, wn1, b1, ws2, wn2, b2,
           wa1, ba1, wa2, ba2, wb1, bb1, wb2, bb2):
    w1 = jnp.concatenate([ws1, wn1], axis=1)
    w2 = jnp.concatenate([ws2, wn2], axis=1)
    wh = jnp.concatenate([wa1, wb1], axis=1)
    h = wa2.shape[0]
    zf = jnp.zeros((h, 1), jnp.float32)
    zt = jnp.zeros((h, 2), jnp.float32)
    wf = jnp.concatenate([
        jnp.concatenate([wa2, zf], axis=1),
        jnp.concatenate([zt, wb2], axis=1),
    ], axis=0)
    bh = jnp.concatenate([ba1, bb1], axis=1)
    bf_ = jnp.concatenate([ba2, bb2], axis=1)
    oa, ob = _forward(x, adj, w1, w2, wh, wf, b1, b2, bh, bf_)
    return {"am1-charges": oa, "am1-wbo-like": ob}


# speedup vs baseline: 2.4529x; 1.3350x over previous
import functools

import jax
import jax.numpy as jnp
from jax.experimental import pallas as pl
from jax.experimental.pallas import tpu as pltpu

_TILE = 128
_GROUP = 8


def _gcn_kernel(x_ref, a_ref, w1_ref, w2_ref, wh_ref, wf_ref,
                b1_ref, b2_ref, bh_ref, bf_ref, o_ref):
    t, n, f_in = x_ref.shape
    r = t * n
    p = _GROUP * n
    n_groups = t // _GROUP

    xb = x_ref[...].reshape(r, f_in)
    a = a_ref[...].reshape(r, n)

    deg = jnp.sum(a, axis=1, keepdims=True)
    rnorm = jnp.where(deg > 0.0, 1.0 / deg, 0.0)
    an = a * rnorm

    rows = jax.lax.broadcasted_iota(jnp.int32, (p, p), 0) // n
    cols = jax.lax.broadcasted_iota(jnp.int32, (p, p), 1) // n
    same = (rows == cols).astype(jnp.float32)

    tj = jax.lax.broadcasted_iota(jnp.int32, (n, p), 0)
    tc = jax.lax.broadcasted_iota(jnp.int32, (n, p), 1)
    tile_mat = (tc % n == tj).astype(jnp.float32)

    blocks = []
    for gi in range(n_groups):
        g = an[gi * p:(gi + 1) * p, :]
        rep = jnp.dot(g, tile_mat, preferred_element_type=jnp.float32)
        blocks.append(rep * same)

    def agg(feat):
        outs = [jnp.dot(blocks[gi], feat[gi * p:(gi + 1) * p, :],
                        preferred_element_type=jnp.float32)
                for gi in range(n_groups)]
        return jnp.concatenate(outs, axis=0)

    y1 = jnp.dot(xb, w1_ref[...], preferred_element_type=jnp.float32)
    n1 = agg(y1[:, f_in:])
    h1 = jnp.maximum(y1[:, :f_in] + n1 + b1_ref[...], 0.0)

    y2 = jnp.dot(h1, w2_ref[...], preferred_element_type=jnp.float32)
    n2 = agg(y2[:, f_in:])
    h2 = jnp.maximum(y2[:, :f_in] + n2 + b2_ref[...], 0.0)

    u = jnp.dot(h2, wh_ref[...], preferred_element_type=jnp.float32) + bh_ref[...]
    ub = jnp.maximum(u, 0.0)
    fin_t = jax.lax.dot_general(
        wf_ref[...], ub, (((1,), (1,)), ((), ())),
        preferred_element_type=jnp.float32) + bf_ref[...]
    o_ref[...] = fin_t


@functools.partial(jax.jit, static_argnames=())
def _forward(x, adj, w1, w2, wh, wf, b1, b2, bh, bf):
    b, n, f_in = x.shape
    t = _TILE
    return pl.pallas_call(
        _gcn_kernel,
        out_shape=jax.ShapeDtypeStruct((3, b * n), jnp.float32),
        grid=(b // t,),
        in_specs=[
            pl.BlockSpec((t, n, f_in), lambda i: (i, 0, 0)),
            pl.BlockSpec((t, n, n), lambda i: (i, 0, 0)),
            pl.BlockSpec(w1.shape, lambda i: (0, 0)),
            pl.BlockSpec(w2.shape, lambda i: (0, 0)),
            pl.BlockSpec(wh.shape, lambda i: (0, 0)),
            pl.BlockSpec(wf.shape, lambda i: (0, 0)),
            pl.BlockSpec(b1.shape, lambda i: (0, 0)),
            pl.BlockSpec(b2.shape, lambda i: (0, 0)),
            pl.BlockSpec(bh.shape, lambda i: (0, 0)),
            pl.BlockSpec(bf.shape, lambda i: (0, 0)),
        ],
        out_specs=pl.BlockSpec((3, t * n), lambda i: (0, i)),
        compiler_params=pltpu.CompilerParams(
            dimension_semantics=("parallel",),
            vmem_limit_bytes=64 * 1024 * 1024,
        ),
    )(x, adj, w1, w2, wh, wf, b1, b2, bh, bf)


def kernel(x, adj, ws1, wn1, b1, ws2, wn2, b2,
           wa1, ba1, wa2, ba2, wb1, bb1, wb2, bb2):
    w1 = jnp.concatenate([ws1, wn1], axis=1)
    w2 = jnp.concatenate([ws2, wn2], axis=1)
    wh = jnp.concatenate([wa1, wb1], axis=1)
    h = wa2.shape[0]
    zf = jnp.zeros((h, 1), jnp.float32)
    zt = jnp.zeros((h, 2), jnp.float32)
    wf = jnp.concatenate([
        jnp.concatenate([wa2, zf], axis=1),
        jnp.concatenate([zt, wb2], axis=1),
    ], axis=0).T
    bh = jnp.concatenate([ba1, bb1], axis=1)
    bf_ = jnp.concatenate([ba2, bb2], axis=1).T
    fin_t = _forward(x, adj, w1, w2, wh, wf, b1, b2, bh, bf_)
    b, n = x.shape[0], x.shape[1]
    oa = jnp.transpose(fin_t[0:2].reshape(2, b, n), (1, 2, 0))
    ob = jnp.transpose(fin_t[2:3].reshape(1, b, n), (1, 2, 0))
    return {"am1-charges": oa, "am1-wbo-like": ob}


# speedup vs baseline: 2.5554x; 1.0418x over previous
import functools

import jax
import jax.numpy as jnp
from jax.experimental import pallas as pl
from jax.experimental.pallas import tpu as pltpu

_TILE = 128
_GROUP = 8


def _gcn_kernel(x_ref, a_ref, w1_ref, w2_ref, wh_ref, wf_ref,
                b1_ref, b2_ref, bh_ref, bf_ref, o_ref):
    t, n, f_in = x_ref.shape
    r = t * n
    p = _GROUP * n
    n_groups = t // _GROUP

    xb = x_ref[...].reshape(r, f_in)
    a = a_ref[...].reshape(r, n).astype(jnp.float32)

    deg = jnp.sum(a, axis=1, keepdims=True)
    rnorm = jnp.where(deg > 0.0, 1.0 / deg, 0.0)
    an = a * rnorm

    rows = jax.lax.broadcasted_iota(jnp.int32, (p, p), 0) // n
    cols = jax.lax.broadcasted_iota(jnp.int32, (p, p), 1) // n
    same = (rows == cols).astype(jnp.float32)

    tj = jax.lax.broadcasted_iota(jnp.int32, (n, p), 0)
    tc = jax.lax.broadcasted_iota(jnp.int32, (n, p), 1)
    tile_mat = (tc % n == tj).astype(jnp.float32)

    blocks = []
    for gi in range(n_groups):
        g = an[gi * p:(gi + 1) * p, :]
        rep = jnp.dot(g, tile_mat, preferred_element_type=jnp.float32)
        blocks.append(rep * same)

    def agg(feat):
        outs = [jnp.dot(blocks[gi], feat[gi * p:(gi + 1) * p, :],
                        preferred_element_type=jnp.float32)
                for gi in range(n_groups)]
        return jnp.concatenate(outs, axis=0)

    y1 = jnp.dot(xb, w1_ref[...], preferred_element_type=jnp.float32)
    n1 = agg(y1[:, f_in:])
    h1 = jnp.maximum(y1[:, :f_in] + n1 + b1_ref[...], 0.0)

    y2 = jnp.dot(h1, w2_ref[...], preferred_element_type=jnp.float32)
    n2 = agg(y2[:, f_in:])
    h2 = jnp.maximum(y2[:, :f_in] + n2 + b2_ref[...], 0.0)

    u = jnp.dot(h2, wh_ref[...], preferred_element_type=jnp.float32) + bh_ref[...]
    ub = jnp.maximum(u, 0.0)
    fin_t = jax.lax.dot_general(
        wf_ref[...], ub, (((1,), (1,)), ((), ())),
        preferred_element_type=jnp.float32) + bf_ref[...]
    o_ref[...] = fin_t


@functools.partial(jax.jit, static_argnames=())
def _forward(x, adj, w1, w2, wh, wf, b1, b2, bh, bf):
    b, n, f_in = x.shape
    t = _TILE
    return pl.pallas_call(
        _gcn_kernel,
        out_shape=jax.ShapeDtypeStruct((3, b * n), jnp.float32),
        grid=(b // t,),
        in_specs=[
            pl.BlockSpec((t, n, f_in), lambda i: (i, 0, 0)),
            pl.BlockSpec((t, n, n), lambda i: (i, 0, 0)),
            pl.BlockSpec(w1.shape, lambda i: (0, 0)),
            pl.BlockSpec(w2.shape, lambda i: (0, 0)),
            pl.BlockSpec(wh.shape, lambda i: (0, 0)),
            pl.BlockSpec(wf.shape, lambda i: (0, 0)),
            pl.BlockSpec(b1.shape, lambda i: (0, 0)),
            pl.BlockSpec(b2.shape, lambda i: (0, 0)),
            pl.BlockSpec(bh.shape, lambda i: (0, 0)),
            pl.BlockSpec(bf.shape, lambda i: (0, 0)),
        ],
        out_specs=pl.BlockSpec((3, t * n), lambda i: (0, i)),
        compiler_params=pltpu.CompilerParams(
            dimension_semantics=("parallel",),
            vmem_limit_bytes=64 * 1024 * 1024,
        ),
    )(x, adj, w1, w2, wh, wf, b1, b2, bh, bf)


def kernel(x, adj, ws1, wn1, b1, ws2, wn2, b2,
           wa1, ba1, wa2, ba2, wb1, bb1, wb2, bb2):
    w1 = jnp.concatenate([ws1, wn1], axis=1)
    w2 = jnp.concatenate([ws2, wn2], axis=1)
    wh = jnp.concatenate([wa1, wb1], axis=1)
    h = wa2.shape[0]
    zf = jnp.zeros((h, 1), jnp.float32)
    zt = jnp.zeros((h, 2), jnp.float32)
    wf = jnp.concatenate([
        jnp.concatenate([wa2, zf], axis=1),
        jnp.concatenate([zt, wb2], axis=1),
    ], axis=0).T
    bh = jnp.concatenate([ba1, bb1], axis=1)
    bf_ = jnp.concatenate([ba2, bb2], axis=1).T
    adj_bf = adj.astype(jnp.bfloat16)
    fin_t = _forward(x, adj_bf, w1, w2, wh, wf, b1, b2, bh, bf_)
    b, n = x.shape[0], x.shape[1]
    oa = jnp.transpose(fin_t[0:2].reshape(2, b, n), (1, 2, 0))
    ob = jnp.transpose(fin_t[2:3].reshape(1, b, n), (1, 2, 0))
    return {"am1-charges": oa, "am1-wbo-like": ob}


# speedup vs baseline: 2.5891x; 1.0132x over previous
import functools

import jax
import jax.numpy as jnp
from jax.experimental import pallas as pl
from jax.experimental.pallas import tpu as pltpu

_TILE = 256
_GROUP = 8


def _gcn_kernel(x_ref, a_ref, w1_ref, w2_ref, wh_ref, wf_ref,
                b1_ref, b2_ref, bh_ref, bf_ref, o_ref):
    t, n, f_in = x_ref.shape
    r = t * n
    p = _GROUP * n
    n_groups = t // _GROUP

    xb = x_ref[...].reshape(r, f_in)
    a = a_ref[...].reshape(r, n).astype(jnp.float32)

    deg = jnp.sum(a, axis=1, keepdims=True)
    rnorm = jnp.where(deg > 0.0, 1.0 / deg, 0.0)
    an = a * rnorm

    rows = jax.lax.broadcasted_iota(jnp.int32, (p, p), 0) // n
    cols = jax.lax.broadcasted_iota(jnp.int32, (p, p), 1) // n
    same = (rows == cols).astype(jnp.float32)

    tj = jax.lax.broadcasted_iota(jnp.int32, (n, p), 0)
    tc = jax.lax.broadcasted_iota(jnp.int32, (n, p), 1)
    tile_mat = (tc % n == tj).astype(jnp.float32)

    blocks = []
    for gi in range(n_groups):
        g = an[gi * p:(gi + 1) * p, :]
        rep = jnp.dot(g, tile_mat, preferred_element_type=jnp.float32)
        blocks.append(rep * same)

    def agg(feat):
        outs = [jnp.dot(blocks[gi], feat[gi * p:(gi + 1) * p, :],
                        preferred_element_type=jnp.float32)
                for gi in range(n_groups)]
        return jnp.concatenate(outs, axis=0)

    y1 = jnp.dot(xb, w1_ref[...], preferred_element_type=jnp.float32)
    n1 = agg(y1[:, f_in:])
    h1 = jnp.maximum(y1[:, :f_in] + n1 + b1_ref[...], 0.0)

    y2 = jnp.dot(h1, w2_ref[...], preferred_element_type=jnp.float32)
    n2 = agg(y2[:, f_in:])
    h2 = jnp.maximum(y2[:, :f_in] + n2 + b2_ref[...], 0.0)

    u = jnp.dot(h2, wh_ref[...], preferred_element_type=jnp.float32) + bh_ref[...]
    ub = jnp.maximum(u, 0.0)
    fin_t = jax.lax.dot_general(
        wf_ref[...], ub, (((1,), (1,)), ((), ())),
        preferred_element_type=jnp.float32) + bf_ref[...]
    o_ref[...] = fin_t


@functools.partial(jax.jit, static_argnames=())
def _forward(x, adj, w1, w2, wh, wf, b1, b2, bh, bf):
    b, n, f_in = x.shape
    t = _TILE
    return pl.pallas_call(
        _gcn_kernel,
        out_shape=jax.ShapeDtypeStruct((3, b * n), jnp.float32),
        grid=(b // t,),
        in_specs=[
            pl.BlockSpec((t, n, f_in), lambda i: (i, 0, 0)),
            pl.BlockSpec((t, n, n), lambda i: (i, 0, 0)),
            pl.BlockSpec(w1.shape, lambda i: (0, 0)),
            pl.BlockSpec(w2.shape, lambda i: (0, 0)),
            pl.BlockSpec(wh.shape, lambda i: (0, 0)),
            pl.BlockSpec(wf.shape, lambda i: (0, 0)),
            pl.BlockSpec(b1.shape, lambda i: (0, 0)),
            pl.BlockSpec(b2.shape, lambda i: (0, 0)),
            pl.BlockSpec(bh.shape, lambda i: (0, 0)),
            pl.BlockSpec(bf.shape, lambda i: (0, 0)),
        ],
        out_specs=pl.BlockSpec((3, t * n), lambda i: (0, i)),
        compiler_params=pltpu.CompilerParams(
            dimension_semantics=("parallel",),
            vmem_limit_bytes=64 * 1024 * 1024,
        ),
    )(x, adj, w1, w2, wh, wf, b1, b2, bh, bf)


def kernel(x, adj, ws1, wn1, b1, ws2, wn2, b2,
           wa1, ba1, wa2, ba2, wb1, bb1, wb2, bb2):
    w1 = jnp.concatenate([ws1, wn1], axis=1)
    w2 = jnp.concatenate([ws2, wn2], axis=1)
    wh = jnp.concatenate([wa1, wb1], axis=1)
    h = wa2.shape[0]
    zf = jnp.zeros((h, 1), jnp.float32)
    zt = jnp.zeros((h, 2), jnp.float32)
    wf = jnp.concatenate([
        jnp.concatenate([wa2, zf], axis=1),
        jnp.concatenate([zt, wb2], axis=1),
    ], axis=0).T
    bh = jnp.concatenate([ba1, bb1], axis=1)
    bf_ = jnp.concatenate([ba2, bb2], axis=1).T
    adj_bf = adj.astype(jnp.bfloat16)
    fin_t = _forward(x, adj_bf, w1, w2, wh, wf, b1, b2, bh, bf_)
    b, n = x.shape[0], x.shape[1]
    oa = jnp.transpose(fin_t[0:2].reshape(2, b, n), (1, 2, 0))
    ob = jnp.transpose(fin_t[2:3].reshape(1, b, n), (1, 2, 0))
    return {"am1-charges": oa, "am1-wbo-like": ob}
